# baseline (speedup 1.0000x reference)
import functools

import jax
import jax.numpy as jnp
from jax import lax
from jax.experimental import pallas as pl
from jax.experimental.pallas import tpu as pltpu
from jax.experimental.pallas import tpu_sc as plsc

VOCAB = 1000000
TBW = 2048
SPLIT = TBW * 245
EMB = 64
N = 100000

NC = 2
NS = 16
NW = NC * NS
ROWS_PER_W = 3200
CH = 320
NBUF = 3
NCHUNK = ROWS_PER_W // CH
N_PAD = NW * ROWS_PER_W

TC_TILE = 2048


def _tc_repack(WT):
    grid = SPLIT // TBW
    noff = SPLIT // TBW

    def body(wt0_ref, wt1_ref, out_ref):
        x0 = wt0_ref[...]
        x1 = wt1_ref[...]
        xc = jnp.concatenate([x0, x1], axis=0)
        eye = jnp.eye(128, dtype=jnp.float32)
        out_ref[...] = lax.dot_general(
            xc, eye, (((0,), (0,)), ((), ())),
            preferred_element_type=jnp.float32)

    return pl.pallas_call(
        body,
        grid=(grid,),
        in_specs=[
            pl.BlockSpec((EMB, TBW), lambda i: (0, i)),
            pl.BlockSpec(
                (EMB, TBW),
                lambda i: (0, jnp.minimum(i + noff, VOCAB // TBW))),
        ],
        out_specs=pl.BlockSpec((TBW, 128), lambda i: (i, 0)),
        out_shape=jax.ShapeDtypeStruct((SPLIT, 128), jnp.float32),
    )(WT, WT)


def _sc_gather(W2, idxh):
    mesh = plsc.VectorSubcoreMesh(
        core_axis_name="c", subcore_axis_name="s",
        num_cores=NC, num_subcores=NS,
    )

    @functools.partial(
        pl.kernel,
        out_type=jax.ShapeDtypeStruct((N_PAD, 128), jnp.float32),
        mesh=mesh,
        scratch_types=[
            [pltpu.VMEM((CH,), jnp.int32) for _ in range(NBUF)],
            [pltpu.VMEM((CH, 128), jnp.float32) for _ in range(NBUF)],
            [pltpu.SemaphoreType.DMA for _ in range(NBUF)],
            [pltpu.SemaphoreType.DMA for _ in range(NBUF)],
        ],
    )
    def k(w_hbm, i_hbm, o_hbm, idx_vs, rows_vs, gsems, wsems):
        wid = lax.axis_index("s") * NC + lax.axis_index("c")
        base = wid * ROWS_PER_W

        def fire(ci):
            b = ci % NBUF
            off = base + ci * CH
            pltpu.sync_copy(i_hbm.at[pl.ds(off, CH)], idx_vs[b])
            pltpu.async_copy(w_hbm.at[idx_vs[b]], rows_vs[b], gsems[b])

        for ci in range(NBUF):
            fire(ci)
        for ci in range(NCHUNK):
            b = ci % NBUF
            off = base + ci * CH
            pltpu.make_async_copy(
                w_hbm.at[idx_vs[b]], rows_vs[b], gsems[b]).wait()
            pltpu.async_copy(
                rows_vs[b], o_hbm.at[pl.ds(off, CH)], wsems[b])
            nci = ci + NBUF
            if nci < NCHUNK:
                pltpu.make_async_copy(
                    rows_vs[b], o_hbm.at[pl.ds(off, CH)], wsems[b]).wait()
                fire(nci)
        for ci in range(NCHUNK - NBUF, NCHUNK):
            b = ci % NBUF
            off = base + ci * CH
            pltpu.make_async_copy(
                rows_vs[b], o_hbm.at[pl.ds(off, CH)], wsems[b]).wait()

    return k(W2, idxh)


def _tc_bilinear(UG, VG, ui, vi, B00, bu, bv, c11):
    grid = N_PAD // TC_TILE

    def body(ug_ref, vg_ref, ui_ref, vi_ref, b00_ref, bu_ref, bv_ref, c_ref,
             out_ref):
        ug = ug_ref[...]
        vg = vg_ref[...]
        pu1 = (ui_ref[...] >= SPLIT).astype(jnp.float32)
        pv1 = (vi_ref[...] >= SPLIT).astype(jnp.float32)
        pu0, pv0 = 1.0 - pu1, 1.0 - pv1
        uL, uR = ug[:, :EMB], ug[:, EMB:]
        vL, vR = vg[:, :EMB], vg[:, EMB:]
        b00 = b00_ref[...]
        bvr = bv_ref[...]
        cL = jnp.dot(uL, b00, preferred_element_type=jnp.float32) + bvr
        cR = jnp.dot(uR, b00, preferred_element_type=jnp.float32) + bvr
        sLL = jnp.sum(cL * vL, axis=1)
        sLR = jnp.sum(cL * vR, axis=1)
        sRL = jnp.sum(cR * vL, axis=1)
        sRR = jnp.sum(cR * vR, axis=1)
        bur = bu_ref[...]
        tL = jnp.dot(uL, bur, preferred_element_type=jnp.float32)[:, 0]
        tR = jnp.dot(uR, bur, preferred_element_type=jnp.float32)[:, 0]
        out_ref[...] = (
            pu0 * (pv0 * sLL + pv1 * sLR + tL)
            + pu1 * (pv0 * sRL + pv1 * sRR + tR)
            + c_ref[0, 0])

    return pl.pallas_call(
        body,
        grid=(grid,),
        in_specs=[
            pl.BlockSpec((TC_TILE, 128), lambda i: (i, 0)),
            pl.BlockSpec((TC_TILE, 128), lambda i: (i, 0)),
            pl.BlockSpec((TC_TILE,), lambda i: (i,)),
            pl.BlockSpec((TC_TILE,), lambda i: (i,)),
            pl.BlockSpec((EMB, EMB), lambda i: (0, 0)),
            pl.BlockSpec((EMB, 1), lambda i: (0, 0)),
            pl.BlockSpec((1, EMB), lambda i: (0, 0)),
            pl.BlockSpec((1, 1), lambda i: (0, 0)),
        ],
        out_specs=pl.BlockSpec((TC_TILE,), lambda i: (i,)),
        out_shape=jax.ShapeDtypeStruct((N_PAD,), jnp.float32),
    )(UG, VG, ui, vi, B00, bu, bv, c11)


@jax.jit
def kernel(U, V, B, us_ind, vs_ind):
    pad = N_PAD - N
    ui = jnp.concatenate(
        [us_ind.astype(jnp.int32), jnp.zeros((pad,), jnp.int32)])
    vi = jnp.concatenate(
        [vs_ind.astype(jnp.int32), jnp.zeros((pad,), jnp.int32)])
    uih = jnp.where(ui >= SPLIT, ui - SPLIT, ui)
    vih = jnp.where(vi >= SPLIT, vi - SPLIT, vi)

    U2 = _tc_repack(U.T)
    UG = _sc_gather(U2, uih)
    V2 = _tc_repack(V.T)
    VG = _sc_gather(V2, vih)

    B00 = B[:EMB, :EMB]
    bu = B[:EMB, EMB:]
    bv = B[EMB:, :EMB]
    c11 = B[EMB:, EMB:]
    out = _tc_bilinear(UG, VG, ui, vi, B00, bu, bv, c11)
    return out[:N]


# speedup vs baseline: 1.1736x; 1.1736x over previous
import functools

import jax
import jax.numpy as jnp
from jax import lax
from jax.experimental import pallas as pl
from jax.experimental.pallas import tpu as pltpu
from jax.experimental.pallas import tpu_sc as plsc

VOCAB = 1000000
TBW = 8192
SPLIT = TBW * 62
EMB = 64
N = 100000

NC = 2
NS = 16
NW = NC * NS
ROWS_PER_W = 3200
CH = 320
NBUF = 3
NCHUNK = ROWS_PER_W // CH
N_PAD = NW * ROWS_PER_W

TC_TILE = 2048


def _tc_repack(WT):
    grid = SPLIT // TBW
    noff = SPLIT // TBW

    def body(wt0_ref, wt1_ref, out_ref):
        x0 = wt0_ref[...]
        x1 = wt1_ref[...]
        xc = jnp.concatenate([x0, x1], axis=0)
        eye = jnp.eye(128, dtype=jnp.float32)
        out_ref[...] = lax.dot_general(
            xc, eye, (((0,), (0,)), ((), ())),
            preferred_element_type=jnp.float32)

    return pl.pallas_call(
        body,
        grid=(grid,),
        in_specs=[
            pl.BlockSpec((EMB, TBW), lambda i: (0, i)),
            pl.BlockSpec(
                (EMB, TBW),
                lambda i: (0, jnp.minimum(i + noff, VOCAB // TBW))),
        ],
        out_specs=pl.BlockSpec((TBW, 128), lambda i: (i, 0)),
        out_shape=jax.ShapeDtypeStruct((SPLIT, 128), jnp.float32),
    )(WT, WT)


def _sc_gather(W2, idxh):
    mesh = plsc.VectorSubcoreMesh(
        core_axis_name="c", subcore_axis_name="s",
        num_cores=NC, num_subcores=NS,
    )

    @functools.partial(
        pl.kernel,
        out_type=jax.ShapeDtypeStruct((N_PAD, 128), jnp.float32),
        mesh=mesh,
        scratch_types=[
            [pltpu.VMEM((CH,), jnp.int32) for _ in range(NBUF)],
            [pltpu.VMEM((CH, 128), jnp.float32) for _ in range(NBUF)],
            [pltpu.SemaphoreType.DMA for _ in range(NBUF)],
            [pltpu.SemaphoreType.DMA for _ in range(NBUF)],
        ],
    )
    def k(w_hbm, i_hbm, o_hbm, idx_vs, rows_vs, gsems, wsems):
        wid = lax.axis_index("c") * NS + lax.axis_index("s")
        base = wid * ROWS_PER_W

        def fire(ci):
            b = ci % NBUF
            off = base + ci * CH
            pltpu.sync_copy(i_hbm.at[pl.ds(off, CH)], idx_vs[b])
            pltpu.async_copy(w_hbm.at[idx_vs[b]], rows_vs[b], gsems[b])

        for ci in range(NBUF):
            fire(ci)
        for ci in range(NCHUNK):
            b = ci % NBUF
            off = base + ci * CH
            pltpu.make_async_copy(
                w_hbm.at[idx_vs[b]], rows_vs[b], gsems[b]).wait()
            pltpu.async_copy(
                rows_vs[b], o_hbm.at[pl.ds(off, CH)], wsems[b])
            nci = ci + NBUF
            if nci < NCHUNK:
                pltpu.make_async_copy(
                    rows_vs[b], o_hbm.at[pl.ds(off, CH)], wsems[b]).wait()
                fire(nci)
        for ci in range(NCHUNK - NBUF, NCHUNK):
            b = ci % NBUF
            off = base + ci * CH
            pltpu.make_async_copy(
                rows_vs[b], o_hbm.at[pl.ds(off, CH)], wsems[b]).wait()

    return k(W2, idxh)


def _tc_bilinear(UG, VG, ui, vi, B00, bu, bv, c11):
    grid = N_PAD // TC_TILE

    def body(ug_ref, vg_ref, ui_ref, vi_ref, b00_ref, bu_ref, bv_ref, c_ref,
             out_ref):
        ug = ug_ref[...]
        vg = vg_ref[...]
        pu1 = (ui_ref[...] >= SPLIT).astype(jnp.float32)
        pv1 = (vi_ref[...] >= SPLIT).astype(jnp.float32)
        pu0, pv0 = 1.0 - pu1, 1.0 - pv1
        uL, uR = ug[:, :EMB], ug[:, EMB:]
        vL, vR = vg[:, :EMB], vg[:, EMB:]
        b00 = b00_ref[...]
        bvr = bv_ref[...]
        cL = jnp.dot(uL, b00, preferred_element_type=jnp.float32) + bvr
        cR = jnp.dot(uR, b00, preferred_element_type=jnp.float32) + bvr
        ones = jnp.ones((EMB, 1), jnp.float32)

        def rowsum(x):
            return jnp.dot(x, ones, preferred_element_type=jnp.float32)[:, 0]

        sLL = rowsum(cL * vL)
        sLR = rowsum(cL * vR)
        sRL = rowsum(cR * vL)
        sRR = rowsum(cR * vR)
        bur = bu_ref[...]
        tL = jnp.dot(uL, bur, preferred_element_type=jnp.float32)[:, 0]
        tR = jnp.dot(uR, bur, preferred_element_type=jnp.float32)[:, 0]
        out_ref[...] = (
            pu0 * (pv0 * sLL + pv1 * sLR + tL)
            + pu1 * (pv0 * sRL + pv1 * sRR + tR)
            + c_ref[0, 0])

    return pl.pallas_call(
        body,
        grid=(grid,),
        in_specs=[
            pl.BlockSpec((TC_TILE, 128), lambda i: (i, 0)),
            pl.BlockSpec((TC_TILE, 128), lambda i: (i, 0)),
            pl.BlockSpec((TC_TILE,), lambda i: (i,)),
            pl.BlockSpec((TC_TILE,), lambda i: (i,)),
            pl.BlockSpec((EMB, EMB), lambda i: (0, 0)),
            pl.BlockSpec((EMB, 1), lambda i: (0, 0)),
            pl.BlockSpec((1, EMB), lambda i: (0, 0)),
            pl.BlockSpec((1, 1), lambda i: (0, 0)),
        ],
        out_specs=pl.BlockSpec((TC_TILE,), lambda i: (i,)),
        out_shape=jax.ShapeDtypeStruct((N_PAD,), jnp.float32),
    )(UG, VG, ui, vi, B00, bu, bv, c11)


@jax.jit
def kernel(U, V, B, us_ind, vs_ind):
    pad = N_PAD - N
    ui = jnp.concatenate(
        [us_ind.astype(jnp.int32), jnp.zeros((pad,), jnp.int32)])
    vi = jnp.concatenate(
        [vs_ind.astype(jnp.int32), jnp.zeros((pad,), jnp.int32)])
    uih = jnp.where(ui >= SPLIT, ui - SPLIT, ui)
    vih = jnp.where(vi >= SPLIT, vi - SPLIT, vi)

    U2 = _tc_repack(U.T)
    UG = _sc_gather(U2, uih)
    V2 = _tc_repack(V.T)
    VG = _sc_gather(V2, vih)

    B00 = B[:EMB, :EMB]
    bu = B[:EMB, EMB:]
    bv = B[EMB:, :EMB]
    c11 = B[EMB:, EMB:]
    out = _tc_bilinear(UG, VG, ui, vi, B00, bu, bv, c11)
    return out[:N]


# speedup vs baseline: 1.5728x; 1.3401x over previous
import functools

import jax
import jax.numpy as jnp
from jax import lax
from jax.experimental import pallas as pl
from jax.experimental.pallas import tpu as pltpu
from jax.experimental.pallas import tpu_sc as plsc

VOCAB = 1000000
TBW = 8192
SPLIT = TBW * 62
EMB = 64
N = 100000

NC = 2
NS = 16
NW = NC * NS
ROWS_PER_W = 3200
CH = 320
NBUF = 3
NCHUNK = ROWS_PER_W // CH
N_PAD = NW * ROWS_PER_W

TC_TILE = 2048


def _tc_repack(WT):
    grid = SPLIT // TBW
    noff = SPLIT // TBW

    def body(wt0_ref, wt1_ref, out_ref):
        x0 = wt0_ref[...]
        x1 = wt1_ref[...]
        xc = jnp.concatenate([x0, x1], axis=0)
        eye = jnp.eye(128, dtype=jnp.float32)
        out_ref[...] = lax.dot_general(
            xc, eye, (((0,), (0,)), ((), ())),
            preferred_element_type=jnp.float32)

    return pl.pallas_call(
        body,
        grid=(grid,),
        in_specs=[
            pl.BlockSpec((EMB, TBW), lambda i: (0, i)),
            pl.BlockSpec(
                (EMB, TBW),
                lambda i: (0, jnp.minimum(i + noff, VOCAB // TBW))),
        ],
        out_specs=pl.BlockSpec((TBW, 128), lambda i: (i, 0)),
        out_shape=jax.ShapeDtypeStruct((SPLIT, 128), jnp.float32),
    )(WT, WT)


def _sc_gather(W2, idxh):
    mesh = plsc.VectorSubcoreMesh(
        core_axis_name="c", subcore_axis_name="s",
        num_cores=NC, num_subcores=NS,
    )

    @functools.partial(
        pl.kernel,
        out_type=jax.ShapeDtypeStruct((N_PAD, 128), jnp.float32),
        mesh=mesh,
        scratch_types=[
            [pltpu.VMEM((CH,), jnp.int32) for _ in range(NBUF)],
            [pltpu.VMEM((CH, 128), jnp.float32) for _ in range(NBUF)],
            [pltpu.SemaphoreType.DMA for _ in range(NBUF)],
            [pltpu.SemaphoreType.DMA for _ in range(NBUF)],
        ],
    )
    def k(w_hbm, i_hbm, o_hbm, idx_vs, rows_vs, gsems, wsems):
        wid = lax.axis_index("c") * NS + lax.axis_index("s")
        base = wid * ROWS_PER_W

        def fire(ci):
            b = ci % NBUF
            off = base + ci * CH
            pltpu.sync_copy(i_hbm.at[pl.ds(off, CH)], idx_vs[b])
            pltpu.async_copy(w_hbm.at[idx_vs[b]], rows_vs[b], gsems[b])

        for ci in range(NBUF):
            fire(ci)
        for ci in range(NCHUNK):
            b = ci % NBUF
            off = base + ci * CH
            pltpu.make_async_copy(
                w_hbm.at[idx_vs[b]], rows_vs[b], gsems[b]).wait()
            pltpu.async_copy(
                rows_vs[b], o_hbm.at[pl.ds(off, CH)], wsems[b])
            nci = ci + NBUF
            if nci < NCHUNK:
                pltpu.make_async_copy(
                    rows_vs[b], o_hbm.at[pl.ds(off, CH)], wsems[b]).wait()
                fire(nci)
        for ci in range(NCHUNK - NBUF, NCHUNK):
            b = ci % NBUF
            off = base + ci * CH
            pltpu.make_async_copy(
                rows_vs[b], o_hbm.at[pl.ds(off, CH)], wsems[b]).wait()

    return k(W2, idxh)


def _tc_bilinear(UG, VG, ui, vi, B00, bu, bv, c11):
    grid = N_PAD // TC_TILE

    def body(ug_ref, vg_ref, ui_ref, vi_ref, b00_ref, bu_ref, bv_ref, c_ref,
             out_ref):
        eye = jnp.eye(128, dtype=jnp.float32)

        def tr(x):
            return lax.dot_general(
                eye, x, (((1,), (1,)), ((), ())),
                preferred_element_type=jnp.float32)

        u_t = tr(ug_ref[...])
        v_t = tr(vg_ref[...])
        pu = (ui_ref[...] >= SPLIT).astype(jnp.float32).reshape(1, TC_TILE)
        pv = (vi_ref[...] >= SPLIT).astype(jnp.float32).reshape(1, TC_TILE)
        u_sel = u_t[:EMB] + pu * (u_t[EMB:] - u_t[:EMB])
        v_sel = v_t[:EMB] + pv * (v_t[EMB:] - v_t[:EMB])
        c_t = lax.dot_general(
            b00_ref[...], u_sel, (((0,), (0,)), ((), ())),
            preferred_element_type=jnp.float32)
        s = jnp.sum((c_t + bv_ref[...]) * v_sel, axis=0)
        t2 = jnp.sum(bu_ref[...] * u_sel, axis=0)
        out_ref[...] = s + t2 + c_ref[0, 0]

    return pl.pallas_call(
        body,
        grid=(grid,),
        in_specs=[
            pl.BlockSpec((TC_TILE, 128), lambda i: (i, 0)),
            pl.BlockSpec((TC_TILE, 128), lambda i: (i, 0)),
            pl.BlockSpec((TC_TILE,), lambda i: (i,)),
            pl.BlockSpec((TC_TILE,), lambda i: (i,)),
            pl.BlockSpec((EMB, EMB), lambda i: (0, 0)),
            pl.BlockSpec((EMB, 1), lambda i: (0, 0)),
            pl.BlockSpec((EMB, 1), lambda i: (0, 0)),
            pl.BlockSpec((1, 1), lambda i: (0, 0)),
        ],
        out_specs=pl.BlockSpec((TC_TILE,), lambda i: (i,)),
        out_shape=jax.ShapeDtypeStruct((N_PAD,), jnp.float32),
    )(UG, VG, ui, vi, B00, bu, bv, c11)


@jax.jit
def kernel(U, V, B, us_ind, vs_ind):
    pad = N_PAD - N
    ui = jnp.concatenate(
        [us_ind.astype(jnp.int32), jnp.zeros((pad,), jnp.int32)])
    vi = jnp.concatenate(
        [vs_ind.astype(jnp.int32), jnp.zeros((pad,), jnp.int32)])
    uih = jnp.where(ui >= SPLIT, ui - SPLIT, ui)
    vih = jnp.where(vi >= SPLIT, vi - SPLIT, vi)

    U2 = _tc_repack(U.T)
    UG = _sc_gather(U2, uih)
    V2 = _tc_repack(V.T)
    VG = _sc_gather(V2, vih)

    B00 = B[:EMB, :EMB]
    bu = B[:EMB, EMB:]
    bv = B[EMB:, :EMB].T
    c11 = B[EMB:, EMB:]
    out = _tc_bilinear(UG, VG, ui, vi, B00, bu, bv, c11)
    return out[:N]


# speedup vs baseline: 1.6803x; 1.0684x over previous
import functools

import jax
import jax.numpy as jnp
from jax import lax
from jax.experimental import pallas as pl
from jax.experimental.pallas import tpu as pltpu
from jax.experimental.pallas import tpu_sc as plsc

VOCAB = 1000000
TBW = 8192
SPLIT = TBW * 62
EMB = 64
N = 100000

NC = 2
NS = 16
NW = NC * NS
ROWS_PER_W = 3200
CH = 320
NBUF = 3
NCHUNK = ROWS_PER_W // CH
N_PAD = NW * ROWS_PER_W

TC_TILE = 2048


def _tc_repack(WT):
    grid = SPLIT // TBW
    noff = SPLIT // TBW

    def body(wt0_ref, wt1_ref, out_ref):
        x0 = wt0_ref[...]
        x1 = wt1_ref[...]
        xc = jnp.concatenate([x0, x1], axis=0)
        eye = jnp.eye(128, dtype=jnp.float32)
        out_ref[...] = lax.dot_general(
            xc, eye, (((0,), (0,)), ((), ())),
            preferred_element_type=jnp.float32)

    return pl.pallas_call(
        body,
        grid=(grid,),
        in_specs=[
            pl.BlockSpec((EMB, TBW), lambda i: (0, i)),
            pl.BlockSpec(
                (EMB, TBW),
                lambda i: (0, jnp.minimum(i + noff, VOCAB // TBW))),
        ],
        out_specs=pl.BlockSpec((TBW, 128), lambda i: (i, 0)),
        out_shape=jax.ShapeDtypeStruct((SPLIT, 128), jnp.float32),
    )(WT, WT)


def _sc_gather(W2, idxh):
    mesh = plsc.VectorSubcoreMesh(
        core_axis_name="c", subcore_axis_name="s",
        num_cores=NC, num_subcores=NS,
    )

    @functools.partial(
        pl.kernel,
        out_type=jax.ShapeDtypeStruct((N_PAD, 128), jnp.float32),
        mesh=mesh,
        scratch_types=[
            [pltpu.VMEM((CH,), jnp.int32) for _ in range(NBUF)],
            [pltpu.VMEM((CH, 128), jnp.float32) for _ in range(NBUF)],
            [pltpu.SemaphoreType.DMA for _ in range(NBUF)],
            [pltpu.SemaphoreType.DMA for _ in range(NBUF)],
        ],
    )
    def k(w_hbm, i_hbm, o_hbm, idx_vs, rows_vs, gsems, wsems):
        wid = lax.axis_index("c") * NS + lax.axis_index("s")

        def off(ci):
            return (ci * NW + wid) * CH

        def fire(ci):
            b = ci % NBUF
            pltpu.sync_copy(i_hbm.at[pl.ds(off(ci), CH)], idx_vs[b])
            pltpu.async_copy(w_hbm.at[idx_vs[b]], rows_vs[b], gsems[b])

        for ci in range(NBUF):
            fire(ci)
        for ci in range(NCHUNK):
            b = ci % NBUF
            pltpu.make_async_copy(
                w_hbm.at[idx_vs[b]], rows_vs[b], gsems[b]).wait()
            pltpu.async_copy(
                rows_vs[b], o_hbm.at[pl.ds(off(ci), CH)], wsems[b])
            nci = ci + NBUF
            if nci < NCHUNK:
                pltpu.make_async_copy(
                    rows_vs[b], o_hbm.at[pl.ds(off(ci), CH)], wsems[b]).wait()
                fire(nci)
        for ci in range(NCHUNK - NBUF, NCHUNK):
            b = ci % NBUF
            pltpu.make_async_copy(
                rows_vs[b], o_hbm.at[pl.ds(off(ci), CH)], wsems[b]).wait()

    return k(W2, idxh)


def _tc_bilinear(UG, VG, ui, vi, B00, bu, bv, c11):
    grid = N_PAD // TC_TILE

    def body(ug_ref, vg_ref, ui_ref, vi_ref, b00_ref, bu_ref, bv_ref, c_ref,
             out_ref):
        eye = jnp.eye(128, dtype=jnp.float32)

        def tr(x):
            return lax.dot_general(
                eye, x, (((1,), (1,)), ((), ())),
                preferred_element_type=jnp.float32)

        u_t = tr(ug_ref[...])
        v_t = tr(vg_ref[...])
        pu = (ui_ref[...] >= SPLIT).astype(jnp.float32).reshape(1, TC_TILE)
        pv = (vi_ref[...] >= SPLIT).astype(jnp.float32).reshape(1, TC_TILE)
        u_sel = u_t[:EMB] + pu * (u_t[EMB:] - u_t[:EMB])
        v_sel = v_t[:EMB] + pv * (v_t[EMB:] - v_t[:EMB])
        c_t = lax.dot_general(
            b00_ref[...], u_sel, (((0,), (0,)), ((), ())),
            preferred_element_type=jnp.float32)
        s = jnp.sum((c_t + bv_ref[...]) * v_sel, axis=0)
        t2 = jnp.sum(bu_ref[...] * u_sel, axis=0)
        out_ref[...] = s + t2 + c_ref[0, 0]

    return pl.pallas_call(
        body,
        grid=(grid,),
        in_specs=[
            pl.BlockSpec((TC_TILE, 128), lambda i: (i, 0)),
            pl.BlockSpec((TC_TILE, 128), lambda i: (i, 0)),
            pl.BlockSpec((TC_TILE,), lambda i: (i,)),
            pl.BlockSpec((TC_TILE,), lambda i: (i,)),
            pl.BlockSpec((EMB, EMB), lambda i: (0, 0)),
            pl.BlockSpec((EMB, 1), lambda i: (0, 0)),
            pl.BlockSpec((EMB, 1), lambda i: (0, 0)),
            pl.BlockSpec((1, 1), lambda i: (0, 0)),
        ],
        out_specs=pl.BlockSpec((TC_TILE,), lambda i: (i,)),
        out_shape=jax.ShapeDtypeStruct((N_PAD,), jnp.float32),
    )(UG, VG, ui, vi, B00, bu, bv, c11)


@jax.jit
def kernel(U, V, B, us_ind, vs_ind):
    pad = N_PAD - N
    ui = jnp.concatenate(
        [us_ind.astype(jnp.int32), jnp.zeros((pad,), jnp.int32)])
    vi = jnp.concatenate(
        [vs_ind.astype(jnp.int32), jnp.zeros((pad,), jnp.int32)])
    uih = jnp.where(ui >= SPLIT, ui - SPLIT, ui)
    vih = jnp.where(vi >= SPLIT, vi - SPLIT, vi)

    U2 = _tc_repack(U.T)
    UG = _sc_gather(U2, uih)
    V2 = _tc_repack(V.T)
    VG = _sc_gather(V2, vih)

    B00 = B[:EMB, :EMB]
    bu = B[:EMB, EMB:]
    bv = B[EMB:, :EMB].T
    c11 = B[EMB:, EMB:]
    out = _tc_bilinear(UG, VG, ui, vi, B00, bu, bv, c11)
    return out[:N]
